# baseline (speedup 1.0000x reference)
import functools

import jax
import jax.numpy as jnp
from jax import lax
from jax.experimental import pallas as pl
from jax.experimental.pallas import tpu as pltpu
from jax.experimental.pallas import tpu_sc as plsc

NUM_USERS = 1000000
NUM_ITEMS = 1000000
EMBED_K = 16
BATCH = 16384

_NC = 2
_NS = 16
_NW = _NC * _NS
_BPW = BATCH // _NW
_L = 16
_RPG = 128 // EMBED_K
_CHUNK = 256


def _body(uidx_hbm, vidx_hbm, w_hbm, h_hbm, out_hbm,
          uidx_v, vidx_v, ugidx_v, vgidx_v, ugrp_v, vgrp_v, out_v,
          sem_u, sem_v):
    wid = lax.axis_index("s") * _NC + lax.axis_index("c")
    base = wid * _BPW

    pltpu.sync_copy(uidx_hbm.at[pl.ds(base, _BPW)], uidx_v)
    pltpu.sync_copy(vidx_hbm.at[pl.ds(base, _BPW)], vidx_v)

    def gidx(i, _):
        u = uidx_v[pl.ds(i * _L, _L)]
        v = vidx_v[pl.ds(i * _L, _L)]
        ugidx_v[pl.ds(i * _L, _L)] = u // _RPG
        vgidx_v[pl.ds(i * _L, _L)] = v // _RPG
        return 0

    lax.fori_loop(0, _BPW // _L, gidx, 0)

    lanes = lax.iota(jnp.int32, _L)

    def chunk(c, _):
        cp_u = pltpu.make_async_copy(
            w_hbm.at[ugidx_v.at[pl.ds(c * _CHUNK, _CHUNK)]], ugrp_v, sem_u)
        cp_v = pltpu.make_async_copy(
            h_hbm.at[vgidx_v.at[pl.ds(c * _CHUNK, _CHUNK)]], vgrp_v, sem_v)
        cp_u.start()
        cp_v.start()
        cp_u.wait()
        cp_v.wait()

        def group(g, _):
            i = c * _CHUNK + g * _L
            rows = g * _L + lanes
            uoff = (uidx_v[pl.ds(i, _L)] % _RPG) * EMBED_K
            voff = (vidx_v[pl.ds(i, _L)] % _RPG) * EMBED_K
            acc = jnp.zeros((_L,), jnp.float32)
            for k in range(EMBED_K):
                u = plsc.load_gather(ugrp_v, [rows, uoff + k])
                v = plsc.load_gather(vgrp_v, [rows, voff + k])
                acc = acc + u * v
            inner = 1.0 / (1.0 + jnp.exp(-acc))
            pred = 1.0 / (1.0 + jnp.exp(-inner))
            out_v[pl.ds(i, _L)] = pred
            return 0

        lax.fori_loop(0, _CHUNK // _L, group, 0)
        return 0

    lax.fori_loop(0, _BPW // _CHUNK, chunk, 0)

    pltpu.sync_copy(out_v, out_hbm.at[pl.ds(base, _BPW)])


@jax.jit
def _run(uidx, vidx, w, h):
    mesh = plsc.VectorSubcoreMesh(core_axis_name="c", subcore_axis_name="s")
    f = pl.kernel(
        _body,
        mesh=mesh,
        out_type=jax.ShapeDtypeStruct((BATCH,), jnp.float32),
        compiler_params=pltpu.CompilerParams(needs_layout_passes=False),
        scratch_types=[
            pltpu.VMEM((_BPW,), jnp.int32),
            pltpu.VMEM((_BPW,), jnp.int32),
            pltpu.VMEM((_BPW,), jnp.int32),
            pltpu.VMEM((_BPW,), jnp.int32),
            pltpu.VMEM((_CHUNK, 128), jnp.float32),
            pltpu.VMEM((_CHUNK, 128), jnp.float32),
            pltpu.VMEM((_BPW,), jnp.float32),
            pltpu.SemaphoreType.DMA,
            pltpu.SemaphoreType.DMA,
        ],
    )
    return f(uidx, vidx, w, h)


def kernel(x, W, H):
    uidx = x[:, 0]
    vidx = x[:, 1]
    one = lax.optimization_barrier(jnp.float32(1.0))
    w = W.reshape(NUM_USERS // _RPG, 128) * one
    h = H.reshape(NUM_ITEMS // _RPG, 128) * one
    return _run(uidx, vidx, w, h)


# speedup vs baseline: 1.5491x; 1.5491x over previous
import functools

import jax
import jax.numpy as jnp
from jax import lax
from jax.experimental import pallas as pl
from jax.experimental.pallas import tpu as pltpu
from jax.experimental.pallas import tpu_sc as plsc

NUM_USERS = 1000000
NUM_ITEMS = 1000000
EMBED_K = 16
BATCH = 16384

_NC = 2
_NS = 16
_NW = _NC * _NS
_BPW = BATCH // _NW
_L = 16
_RPG = 128 // EMBED_K
_CHUNK = 256

_WCOLS = 1536
_NWIN = 651
_WPT = 41
_WROWS = _WCOLS // _RPG
_MAIN = _NWIN * _WCOLS


def _fold_window(src_v, dst_v, lanes):
    def fold(r, _):
        for j in range(_RPG):
            col = r * _RPG + j
            vals = plsc.load_gather(src_v, [lanes, jnp.full((_L,), col, jnp.int32)])
            dst_v[r, pl.ds(j * _L, _L)] = vals
        return 0

    lax.fori_loop(0, _WROWS, fold, 0)


def _convert_body(wt_hbm, ht_hbm, wrm_hbm, hrm_hbm,
                  in0_v, in1_v, out0_v, out1_v, tail_v, sem_r, sem_w):
    wid = lax.axis_index("s") * _NC + lax.axis_index("c")
    local = wid // 2
    lanes = lax.iota(jnp.int32, _L)

    in_bufs = (in0_v, in1_v)
    out_bufs = (out0_v, out1_v)

    def run_table(src_hbm, dst_hbm):
        def rd(i, parity):
            w = local + i * 16
            return pltpu.make_async_copy(
                src_hbm.at[:, pl.ds(w * _WCOLS, _WCOLS)], in_bufs[parity],
                sem_r)

        def wr(i, parity):
            w = local + i * 16
            return pltpu.make_async_copy(
                out_bufs[parity],
                dst_hbm.at[pl.ds(w * _WROWS, _WROWS), :], sem_w)

        def win_ok(i):
            return local + i * 16 < _NWIN

        @pl.when(win_ok(0))
        def _():
            rd(0, 0).start()

        def step(t, _):
            for p in range(2):
                i = t * 2 + p

                @pl.when(win_ok(i))
                def _(i=i, p=p):
                    rd(i, p).wait()

                    @pl.when(win_ok(i + 1))
                    def _(i=i, p=p):
                        rd(i + 1, 1 - p).start()

                    @pl.when((i >= 2) & win_ok(i))
                    def _(i=i, p=p):
                        wr(i - 2, p).wait()

                    _fold_window(in_bufs[p], out_bufs[p], lanes)
                    wr(i, p).start()

            return 0

        lax.fori_loop(0, (_WPT + 1) // 2, step, 0)

        for _i in range(2):
            pltpu.make_async_copy(
                out0_v, dst_hbm.at[pl.ds(0, _WROWS), :], sem_w).wait()

        @pl.when(local == 0)
        def _():
            pltpu.sync_copy(src_hbm.at[:, pl.ds(_MAIN, 64)], tail_v)

            def fold_tail(r, _):
                for j in range(_RPG):
                    col = r * _RPG + j
                    vals = plsc.load_gather(
                        tail_v, [lanes, jnp.full((_L,), col, jnp.int32)])
                    out0_v[r, pl.ds(j * _L, _L)] = vals
                return 0

            lax.fori_loop(0, 64 // _RPG, fold_tail, 0)
            pltpu.sync_copy(out0_v.at[pl.ds(0, 64 // _RPG), :],
                            dst_hbm.at[pl.ds(_NWIN * _WROWS, 64 // _RPG), :])

    @pl.when(wid % 2 == 0)
    def _():
        run_table(wt_hbm, wrm_hbm)

    @pl.when(wid % 2 == 1)
    def _():
        run_table(ht_hbm, hrm_hbm)


def _lookup_body(uidx_hbm, vidx_hbm, w_hbm, h_hbm, out_hbm,
                 uidx_v, vidx_v, ugidx_v, vgidx_v, ugrp_v, vgrp_v, out_v,
                 sem_u, sem_v):
    wid = lax.axis_index("s") * _NC + lax.axis_index("c")
    base = wid * _BPW

    pltpu.sync_copy(uidx_hbm.at[pl.ds(base, _BPW)], uidx_v)
    pltpu.sync_copy(vidx_hbm.at[pl.ds(base, _BPW)], vidx_v)

    def gidx(i, _):
        u = uidx_v[pl.ds(i * _L, _L)]
        v = vidx_v[pl.ds(i * _L, _L)]
        ugidx_v[pl.ds(i * _L, _L)] = u // _RPG
        vgidx_v[pl.ds(i * _L, _L)] = v // _RPG
        return 0

    lax.fori_loop(0, _BPW // _L, gidx, 0)

    lanes = lax.iota(jnp.int32, _L)

    def chunk(c, _):
        cp_u = pltpu.make_async_copy(
            w_hbm.at[ugidx_v.at[pl.ds(c * _CHUNK, _CHUNK)]], ugrp_v, sem_u)
        cp_v = pltpu.make_async_copy(
            h_hbm.at[vgidx_v.at[pl.ds(c * _CHUNK, _CHUNK)]], vgrp_v, sem_v)
        cp_u.start()
        cp_v.start()
        cp_u.wait()
        cp_v.wait()

        def group(g, _):
            i = c * _CHUNK + g * _L
            rows = g * _L + lanes
            uoff = (uidx_v[pl.ds(i, _L)] % _RPG) * EMBED_K
            voff = (vidx_v[pl.ds(i, _L)] % _RPG) * EMBED_K
            acc = jnp.zeros((_L,), jnp.float32)
            for k in range(EMBED_K):
                u = plsc.load_gather(ugrp_v, [rows, uoff + k])
                v = plsc.load_gather(vgrp_v, [rows, voff + k])
                acc = acc + u * v
            inner = 1.0 / (1.0 + jnp.exp(-acc))
            pred = 1.0 / (1.0 + jnp.exp(-inner))
            out_v[pl.ds(i, _L)] = pred
            return 0

        lax.fori_loop(0, _CHUNK // _L, group, 0)
        return 0

    lax.fori_loop(0, _BPW // _CHUNK, chunk, 0)

    pltpu.sync_copy(out_v, out_hbm.at[pl.ds(base, _BPW)])


@jax.jit
def _run(uidx, vidx, wt, ht):
    mesh = plsc.VectorSubcoreMesh(core_axis_name="c", subcore_axis_name="s")
    grp_shape = jax.ShapeDtypeStruct((NUM_USERS // _RPG, 128), jnp.float32)
    convert = pl.kernel(
        _convert_body,
        mesh=mesh,
        out_type=(grp_shape, grp_shape),
        compiler_params=pltpu.CompilerParams(needs_layout_passes=False),
        scratch_types=[
            pltpu.VMEM((EMBED_K, _WCOLS), jnp.float32),
            pltpu.VMEM((EMBED_K, _WCOLS), jnp.float32),
            pltpu.VMEM((_WROWS, 128), jnp.float32),
            pltpu.VMEM((_WROWS, 128), jnp.float32),
            pltpu.VMEM((EMBED_K, 64), jnp.float32),
            pltpu.SemaphoreType.DMA,
            pltpu.SemaphoreType.DMA,
        ],
    )
    wrm, hrm = convert(wt, ht)

    lookup = pl.kernel(
        _lookup_body,
        mesh=mesh,
        out_type=jax.ShapeDtypeStruct((BATCH,), jnp.float32),
        compiler_params=pltpu.CompilerParams(needs_layout_passes=False),
        scratch_types=[
            pltpu.VMEM((_BPW,), jnp.int32),
            pltpu.VMEM((_BPW,), jnp.int32),
            pltpu.VMEM((_BPW,), jnp.int32),
            pltpu.VMEM((_BPW,), jnp.int32),
            pltpu.VMEM((_CHUNK, 128), jnp.float32),
            pltpu.VMEM((_CHUNK, 128), jnp.float32),
            pltpu.VMEM((_BPW,), jnp.float32),
            pltpu.SemaphoreType.DMA,
            pltpu.SemaphoreType.DMA,
        ],
    )
    return lookup(uidx, vidx, wrm, hrm)


def kernel(x, W, H):
    uidx = x[:, 0]
    vidx = x[:, 1]
    return _run(uidx, vidx, W.T, H.T)


# speedup vs baseline: 4.1289x; 2.6654x over previous
import functools

import jax
import jax.numpy as jnp
from jax import lax
from jax.experimental import pallas as pl
from jax.experimental.pallas import tpu as pltpu
from jax.experimental.pallas import tpu_sc as plsc

NUM_USERS = 1000000
NUM_ITEMS = 1000000
EMBED_K = 16
BATCH = 16384

_NC = 2
_NS = 16
_NW = _NC * _NS
_BPW = BATCH // _NW
_L = 16
_RPG = 128 // EMBED_K
_CHUNK = 256

_WCOLS = 1536
_NWIN = 651
_WPT = 41
_WROWS = _WCOLS // _RPG
_MAIN = _NWIN * _WCOLS


def _fold_window(src_v, dst_v, lanes):
    def fold(b, _):
        cbase = b * _L
        for d in range(_L):
            lc = (lanes + d) & (_L - 1)
            vals = plsc.load_gather(src_v, [lanes, cbase + lc])
            rowv = 2 * b + (lc // _RPG)
            colv = (lc % _RPG) * _L + lanes
            plsc.store_scatter(dst_v, [rowv, colv], vals)
        return 0

    lax.fori_loop(0, _WCOLS // _L, fold, 0)


def _convert_body(wt_hbm, ht_hbm, wrm_hbm, hrm_hbm,
                  in0_v, in1_v, out0_v, out1_v, tail_v, sem_r, sem_w):
    wid = lax.axis_index("s") * _NC + lax.axis_index("c")
    local = wid // 2
    lanes = lax.iota(jnp.int32, _L)

    in_bufs = (in0_v, in1_v)
    out_bufs = (out0_v, out1_v)

    def run_table(src_hbm, dst_hbm):
        def rd(i, parity):
            w = local + i * 16
            return pltpu.make_async_copy(
                src_hbm.at[:, pl.ds(w * _WCOLS, _WCOLS)], in_bufs[parity],
                sem_r)

        def wr(i, parity):
            w = local + i * 16
            return pltpu.make_async_copy(
                out_bufs[parity],
                dst_hbm.at[pl.ds(w * _WROWS, _WROWS), :], sem_w)

        def win_ok(i):
            return local + i * 16 < _NWIN

        @pl.when(win_ok(0))
        def _():
            rd(0, 0).start()

        def step(t, _):
            for p in range(2):
                i = t * 2 + p

                @pl.when(win_ok(i))
                def _(i=i, p=p):
                    rd(i, p).wait()

                    @pl.when(win_ok(i + 1))
                    def _(i=i, p=p):
                        rd(i + 1, 1 - p).start()

                    @pl.when((i >= 2) & win_ok(i))
                    def _(i=i, p=p):
                        wr(i - 2, p).wait()

                    _fold_window(in_bufs[p], out_bufs[p], lanes)
                    wr(i, p).start()

            return 0

        lax.fori_loop(0, (_WPT + 1) // 2, step, 0)

        for _i in range(2):
            pltpu.make_async_copy(
                out0_v, dst_hbm.at[pl.ds(0, _WROWS), :], sem_w).wait()

        @pl.when(local == 0)
        def _():
            pltpu.sync_copy(src_hbm.at[:, pl.ds(_MAIN, 64)], tail_v)

            def fold_tail(r, _):
                for j in range(_RPG):
                    col = r * _RPG + j
                    vals = plsc.load_gather(
                        tail_v, [lanes, jnp.full((_L,), col, jnp.int32)])
                    out0_v[r, pl.ds(j * _L, _L)] = vals
                return 0

            lax.fori_loop(0, 64 // _RPG, fold_tail, 0)
            pltpu.sync_copy(out0_v.at[pl.ds(0, 64 // _RPG), :],
                            dst_hbm.at[pl.ds(_NWIN * _WROWS, 64 // _RPG), :])

    @pl.when(wid % 2 == 0)
    def _():
        run_table(wt_hbm, wrm_hbm)

    @pl.when(wid % 2 == 1)
    def _():
        run_table(ht_hbm, hrm_hbm)


def _lookup_body(uidx_hbm, vidx_hbm, w_hbm, h_hbm, out_hbm,
                 uidx_v, vidx_v, ugidx_v, vgidx_v, ugrp_v, vgrp_v, out_v,
                 sem_u, sem_v):
    wid = lax.axis_index("s") * _NC + lax.axis_index("c")
    base = wid * _BPW

    pltpu.sync_copy(uidx_hbm.at[pl.ds(base, _BPW)], uidx_v)
    pltpu.sync_copy(vidx_hbm.at[pl.ds(base, _BPW)], vidx_v)

    def gidx(i, _):
        u = uidx_v[pl.ds(i * _L, _L)]
        v = vidx_v[pl.ds(i * _L, _L)]
        ugidx_v[pl.ds(i * _L, _L)] = u // _RPG
        vgidx_v[pl.ds(i * _L, _L)] = v // _RPG
        return 0

    lax.fori_loop(0, _BPW // _L, gidx, 0)

    lanes = lax.iota(jnp.int32, _L)

    def chunk(c, _):
        cp_u = pltpu.make_async_copy(
            w_hbm.at[ugidx_v.at[pl.ds(c * _CHUNK, _CHUNK)]], ugrp_v, sem_u)
        cp_v = pltpu.make_async_copy(
            h_hbm.at[vgidx_v.at[pl.ds(c * _CHUNK, _CHUNK)]], vgrp_v, sem_v)
        cp_u.start()
        cp_v.start()
        cp_u.wait()
        cp_v.wait()

        def group(g, _):
            i = c * _CHUNK + g * _L
            rows = g * _L + lanes
            uoff = (uidx_v[pl.ds(i, _L)] % _RPG) * EMBED_K
            voff = (vidx_v[pl.ds(i, _L)] % _RPG) * EMBED_K
            acc = jnp.zeros((_L,), jnp.float32)
            for k in range(EMBED_K):
                u = plsc.load_gather(ugrp_v, [rows, uoff + k])
                v = plsc.load_gather(vgrp_v, [rows, voff + k])
                acc = acc + u * v
            inner = 1.0 / (1.0 + jnp.exp(-acc))
            pred = 1.0 / (1.0 + jnp.exp(-inner))
            out_v[pl.ds(i, _L)] = pred
            return 0

        lax.fori_loop(0, _CHUNK // _L, group, 0)
        return 0

    lax.fori_loop(0, _BPW // _CHUNK, chunk, 0)

    pltpu.sync_copy(out_v, out_hbm.at[pl.ds(base, _BPW)])


@jax.jit
def _run(uidx, vidx, wt, ht):
    mesh = plsc.VectorSubcoreMesh(core_axis_name="c", subcore_axis_name="s")
    grp_shape = jax.ShapeDtypeStruct((NUM_USERS // _RPG, 128), jnp.float32)
    convert = pl.kernel(
        _convert_body,
        mesh=mesh,
        out_type=(grp_shape, grp_shape),
        compiler_params=pltpu.CompilerParams(needs_layout_passes=False),
        scratch_types=[
            pltpu.VMEM((EMBED_K, _WCOLS), jnp.float32),
            pltpu.VMEM((EMBED_K, _WCOLS), jnp.float32),
            pltpu.VMEM((_WROWS, 128), jnp.float32),
            pltpu.VMEM((_WROWS, 128), jnp.float32),
            pltpu.VMEM((EMBED_K, 64), jnp.float32),
            pltpu.SemaphoreType.DMA,
            pltpu.SemaphoreType.DMA,
        ],
    )
    wrm, hrm = convert(wt, ht)

    lookup = pl.kernel(
        _lookup_body,
        mesh=mesh,
        out_type=jax.ShapeDtypeStruct((BATCH,), jnp.float32),
        compiler_params=pltpu.CompilerParams(needs_layout_passes=False),
        scratch_types=[
            pltpu.VMEM((_BPW,), jnp.int32),
            pltpu.VMEM((_BPW,), jnp.int32),
            pltpu.VMEM((_BPW,), jnp.int32),
            pltpu.VMEM((_BPW,), jnp.int32),
            pltpu.VMEM((_CHUNK, 128), jnp.float32),
            pltpu.VMEM((_CHUNK, 128), jnp.float32),
            pltpu.VMEM((_BPW,), jnp.float32),
            pltpu.SemaphoreType.DMA,
            pltpu.SemaphoreType.DMA,
        ],
    )
    return lookup(uidx, vidx, wrm, hrm)


def kernel(x, W, H):
    uidx = x[:, 0]
    vidx = x[:, 1]
    return _run(uidx, vidx, W.T, H.T)
